# baseline (speedup 1.0000x reference)
import functools

import jax
import jax.numpy as jnp
from jax import lax
from jax.experimental import pallas as pl
from jax.experimental.pallas import tpu as pltpu
from jax.experimental.pallas import tpu_sc as plsc

N_WORKERS = 32
CHUNK = 128


def kernel(atom_number, embedding_list):
    B, = atom_number.shape
    V, D = embedding_list.shape
    b_per_w = B // N_WORKERS
    n_chunks = b_per_w // CHUNK

    mesh = plsc.VectorSubcoreMesh(core_axis_name="c", subcore_axis_name="s")

    @functools.partial(
        pl.kernel,
        mesh=mesh,
        out_type=jax.ShapeDtypeStruct((B, D), jnp.float32),
        scratch_types=[
            pltpu.VMEM_SHARED((V, D), jnp.float32),
            pltpu.VMEM((n_chunks, CHUNK), jnp.int32),
            pltpu.VMEM((b_per_w, D), jnp.float32),
            pltpu.SemaphoreType.DMA,
            pltpu.SemaphoreType.DMA,
        ],
    )
    def gather_kernel(table_hbm, idx_hbm, out_hbm, table_sp, idx_v, rows_v,
                      gsem, osem):
        s = lax.axis_index("s")
        c = lax.axis_index("c")
        wid = s * 2 + c
        base = wid * b_per_w

        @pl.when(s == 0)
        def _():
            pltpu.sync_copy(table_hbm, table_sp)

        for j in range(n_chunks):
            pltpu.sync_copy(idx_hbm.at[pl.ds(base + j * CHUNK, CHUNK)],
                            idx_v.at[j])

        plsc.subcore_barrier()

        gathers = [
            pltpu.async_copy(
                table_sp.at[idx_v.at[j]],
                rows_v.at[pl.ds(j * CHUNK, CHUNK)],
                gsem,
            )
            for j in range(n_chunks)
        ]
        outs = []
        for j in range(n_chunks):
            gathers[j].wait()
            outs.append(
                pltpu.async_copy(
                    rows_v.at[pl.ds(j * CHUNK, CHUNK)],
                    out_hbm.at[pl.ds(base + j * CHUNK, CHUNK)],
                    osem,
                )
            )
        for o in outs:
            o.wait()

    return gather_kernel(embedding_list, atom_number)


# speedup vs baseline: 1.0562x; 1.0562x over previous
import functools

import jax
import jax.numpy as jnp
from jax import lax
from jax.experimental import pallas as pl
from jax.experimental.pallas import tpu as pltpu
from jax.experimental.pallas import tpu_sc as plsc

N_WORKERS = 32
CHUNK = 64


def kernel(atom_number, embedding_list):
    B, = atom_number.shape
    V, D = embedding_list.shape
    b_per_w = B // N_WORKERS
    n_chunks = b_per_w // CHUNK

    idx3 = atom_number.astype(jnp.int32).reshape(N_WORKERS, n_chunks, CHUNK)

    mesh = plsc.VectorSubcoreMesh(core_axis_name="c", subcore_axis_name="s")

    @functools.partial(
        pl.kernel,
        mesh=mesh,
        out_type=jax.ShapeDtypeStruct((B, D), jnp.float32),
        scratch_types=[
            pltpu.VMEM_SHARED((V, D), jnp.float32),
            pltpu.VMEM((n_chunks, CHUNK), jnp.int32),
            pltpu.VMEM((b_per_w, D), jnp.float32),
            pltpu.SemaphoreType.DMA,
            pltpu.SemaphoreType.DMA,
        ],
    )
    def gather_kernel(table_hbm, idx_hbm, out_hbm, table_sp, idx_v, rows_v,
                      gsem, osem):
        s = lax.axis_index("s")
        c = lax.axis_index("c")
        wid = s * 2 + c

        @pl.when(s == 0)
        def _():
            pltpu.sync_copy(table_hbm, table_sp)

        pltpu.sync_copy(idx_hbm.at[wid], idx_v)
        plsc.subcore_barrier()

        gathers = [
            pltpu.async_copy(
                table_sp.at[idx_v.at[j]],
                rows_v.at[pl.ds(j * CHUNK, CHUNK)],
                gsem,
            )
            for j in range(n_chunks)
        ]
        outs = []
        for j in range(n_chunks):
            gathers[j].wait()
            outs.append(
                pltpu.async_copy(
                    rows_v.at[pl.ds(j * CHUNK, CHUNK)],
                    out_hbm.at[pl.ds(wid * b_per_w + j * CHUNK, CHUNK)],
                    osem,
                )
            )
        for o in outs:
            o.wait()

    return gather_kernel(embedding_list, idx3)


# speedup vs baseline: 1.0600x; 1.0036x over previous
import functools

import jax
import jax.numpy as jnp
from jax import lax
from jax.experimental import pallas as pl
from jax.experimental.pallas import tpu as pltpu
from jax.experimental.pallas import tpu_sc as plsc

N_WORKERS = 32
CHUNK = 64


def kernel(atom_number, embedding_list):
    B, = atom_number.shape
    V, D = embedding_list.shape
    b_per_w = B // N_WORKERS
    n_chunks = b_per_w // CHUNK

    mesh = plsc.VectorSubcoreMesh(core_axis_name="c", subcore_axis_name="s")

    @functools.partial(
        pl.kernel,
        mesh=mesh,
        out_type=jax.ShapeDtypeStruct((B, D), jnp.float32),
        scratch_types=[
            pltpu.VMEM_SHARED((V, D), jnp.float32),
            pltpu.VMEM((b_per_w,), jnp.int32),
            pltpu.VMEM((b_per_w, D), jnp.float32),
            pltpu.SemaphoreType.DMA,
            pltpu.SemaphoreType.DMA,
        ],
    )
    def gather_kernel(table_hbm, idx_hbm, out_hbm, table_sp, idx_v, rows_v,
                      gsem, osem):
        s = lax.axis_index("s")
        c = lax.axis_index("c")
        wid = s * 2 + c

        @pl.when(s == 0)
        def _():
            pltpu.sync_copy(table_hbm, table_sp)

        pltpu.sync_copy(idx_hbm.at[pl.ds(wid * b_per_w, b_per_w)], idx_v)
        plsc.subcore_barrier()

        gathers = [
            pltpu.async_copy(
                table_sp.at[idx_v.at[pl.ds(j * CHUNK, CHUNK)]],
                rows_v.at[pl.ds(j * CHUNK, CHUNK)],
                gsem,
            )
            for j in range(n_chunks)
        ]
        outs = []
        for j in range(n_chunks):
            gathers[j].wait()
            outs.append(
                pltpu.async_copy(
                    rows_v.at[pl.ds(j * CHUNK, CHUNK)],
                    out_hbm.at[pl.ds(wid * b_per_w + j * CHUNK, CHUNK)],
                    osem,
                )
            )
        for o in outs:
            o.wait()

    return gather_kernel(embedding_list, atom_number)
